# Initial kernel scaffold; baseline (speedup 1.0000x reference)
import jax
import jax.numpy as jnp
from jax.experimental import pallas as pl


def kernel(x, pos, edge_index, edge_attr, W1, Wr1, g1, b1, W2, Wr2, g2, b2, Wlin, g3, b3):
    raise NotImplementedError("write your pallas kernel here")


# speedup vs baseline: 1.8772x; 1.8772x over previous
import functools

import jax
import jax.numpy as jnp
from jax import lax
from jax.experimental import pallas as pl
from jax.experimental.pallas import tpu as pltpu
from jax.experimental.pallas import tpu_sc as plsc

N = 10000
E = 320000
FP = 130
C = 128
M = 5
K = M * M

NC, NS, L = 2, 16, 16
NW = NC * NS
EW = E // NW
CH = 40
NCH = EW // CH
RPT = N // NS
D1 = C + L


def _prep_body(src_ref, ea0_ref, ea1_ref, gidx_ref, b_ref):
    src = src_ref[...]
    v0 = ea0_ref[...] * float(M - 1)
    v1 = ea1_ref[...] * float(M - 1)
    i0 = jnp.clip(jnp.floor(v0), 0.0, float(M - 2))
    i1 = jnp.clip(jnp.floor(v1), 0.0, float(M - 2))
    f0 = v0 - i0
    f1 = v1 - i1
    base = (i0.astype(jnp.int32) * M + i1.astype(jnp.int32)) * N + src
    gidx_ref[0] = base
    gidx_ref[1] = base + N
    gidx_ref[2] = base + M * N
    gidx_ref[3] = base + (M + 1) * N
    b_ref[0] = (1.0 - f0) * (1.0 - f1)
    b_ref[1] = (1.0 - f0) * f1
    b_ref[2] = f0 * (1.0 - f1)
    b_ref[3] = f0 * f1


def _edge_prep(src2d, ea0, ea1):
    r, c = src2d.shape
    return pl.pallas_call(
        _prep_body,
        out_shape=[jax.ShapeDtypeStruct((4, r, c), jnp.int32),
                   jax.ShapeDtypeStruct((4, r, c), jnp.float32)],
    )(src2d, ea0, ea1)


def _tk_body(x_ref, w_ref, out_ref):
    out_ref[0] = jnp.dot(x_ref[...], w_ref[0],
                         preferred_element_type=jnp.float32)


def _t_build(x, W):
    k, f, c = W.shape
    n = x.shape[0]
    return pl.pallas_call(
        _tk_body,
        grid=(k,),
        in_specs=[pl.BlockSpec((n, f), lambda i: (0, 0)),
                  pl.BlockSpec((1, f, c), lambda i: (i, 0, 0))],
        out_specs=pl.BlockSpec((1, n, c), lambda i: (i, 0, 0)),
        out_shape=jax.ShapeDtypeStruct((k, n, c), jnp.float32),
    )(x, W)


def _bn(v, g, b):
    mu = jnp.mean(v, axis=0, keepdims=True)
    var = jnp.mean((v - mu) ** 2, axis=0, keepdims=True)
    return (v - mu) * lax.rsqrt(var + 1e-5) * g + b


def _post1_body(acc_ref, cnt_ref, xin_ref, wr_ref, g_ref, b_ref, h_ref):
    cnt = jnp.maximum(cnt_ref[0] + cnt_ref[1], 1.0)
    conv = (acc_ref[0] + acc_ref[1]) / cnt + jnp.dot(
        xin_ref[...], wr_ref[...], preferred_element_type=jnp.float32)
    h_ref[...] = jnp.maximum(_bn(conv, g_ref[...], b_ref[...]), 0.0)


def _post1(acc, cnt, xin, wr, g, b):
    return pl.pallas_call(
        _post1_body,
        out_shape=jax.ShapeDtypeStruct((N, C), jnp.float32),
    )(acc, cnt, xin, wr, g.reshape(1, C), b.reshape(1, C))


def _post2_body(acc_ref, cnt_ref, h_ref, wr_ref, xin_ref, wlin_ref,
                g2_ref, b2_ref, g3_ref, b3_ref, out_ref):
    cnt = jnp.maximum(cnt_ref[0] + cnt_ref[1], 1.0)
    conv = (acc_ref[0] + acc_ref[1]) / cnt + jnp.dot(
        h_ref[...], wr_ref[...], preferred_element_type=jnp.float32)
    y = _bn(conv, g2_ref[...], b2_ref[...])
    sk = _bn(jnp.dot(xin_ref[...], wlin_ref[...],
                     preferred_element_type=jnp.float32),
             g3_ref[...], b3_ref[...])
    out_ref[...] = jnp.maximum(y + sk, 0.0)


def _post2(acc2, cnt, h, wr2, xin, wlin, g2, b2, g3, b3):
    return pl.pallas_call(
        _post2_body,
        out_shape=jax.ShapeDtypeStruct((N, C), jnp.float32),
    )(acc2, cnt, h, wr2, xin, wlin,
      g2.reshape(1, C), b2.reshape(1, C), g3.reshape(1, C), b3.reshape(1, C))


NR = 80
CHD = 2000


def _sc_cnt(dst):
    mesh = plsc.VectorSubcoreMesh(core_axis_name="c", subcore_axis_name="s",
                                  num_cores=NC, num_subcores=NS)

    @functools.partial(
        pl.kernel,
        out_type=jax.ShapeDtypeStruct((NC, NR, C), jnp.float32),
        mesh=mesh,
        compiler_params=pltpu.CompilerParams(needs_layout_passes=False),
        scratch_types=[
            pltpu.VMEM((CHD,), jnp.int32),
            pltpu.VMEM((NR, C), jnp.float32),
            pltpu.VMEM((NR,), jnp.int32),
            pltpu.VMEM_SHARED((NR, C), jnp.float32),
            pltpu.SemaphoreType.DMA,
        ],
    )
    def cnt_k(dst_ref, cnt_out, dst_v, cnt_v, rid_v, cnt_sh, sem):
        cid = lax.axis_index("c")
        sid = lax.axis_index("s")
        wid = cid * NS + sid
        zv = jnp.zeros((L,), jnp.float32)
        iv = lax.iota(jnp.int32, L)
        ones = jnp.ones((L,), jnp.float32)

        @pl.loop(0, NR)
        def _(r):
            for v in range(C // L):
                cnt_v[r, pl.ds(v * L, L)] = zv

        @pl.loop(0, NR // L)
        def _(g):
            rid_v[pl.ds(g * L, L)] = iv + g * L

        @pl.when(sid == 0)
        def _():
            pltpu.sync_copy(cnt_v, cnt_sh)

        plsc.subcore_barrier()

        ebase = wid * EW

        @pl.loop(0, EW // CHD)
        def _(i):
            pltpu.async_copy(dst_ref.at[pl.ds(ebase + i * CHD, CHD)],
                             dst_v, sem).wait()

            @pl.loop(0, CHD // L)
            def _(g):
                dv = dst_v[pl.ds(g * L, L)]
                plsc.addupdate_scatter(
                    cnt_v, [lax.shift_right_logical(dv, 7),
                            jnp.bitwise_and(dv, 127)], ones)

        pltpu.async_copy(cnt_v, cnt_sh.at[rid_v], sem, add=True).wait()
        plsc.subcore_barrier()

        @pl.when(sid == 0)
        def _():
            pltpu.sync_copy(cnt_sh, cnt_out.at[cid])

    return cnt_k(dst)


def _sc_conv(t_flat, gidx, dst, b4, zeros):
    mesh = plsc.VectorSubcoreMesh(core_axis_name="c", subcore_axis_name="s",
                                  num_cores=NC, num_subcores=NS)

    @functools.partial(
        pl.kernel,
        out_type=jax.ShapeDtypeStruct((NC, N, C), jnp.float32),
        mesh=mesh,
        compiler_params=pltpu.CompilerParams(needs_layout_passes=False),
        scratch_types=[
            pltpu.VMEM((4, CH), jnp.int32),
            pltpu.VMEM((CH,), jnp.int32),
            pltpu.VMEM((4 * CH,), jnp.float32),
            pltpu.VMEM((4, CH, C), jnp.float32),
            pltpu.VMEM_SHARED((N, C), jnp.float32),
            pltpu.SemaphoreType.DMA,
            pltpu.SemaphoreType.DMA,
            pltpu.SemaphoreType.DMA,
        ],
    )
    def conv(t_ref, gidx_ref, dst_ref, b_ref, z_ref, out_ref,
             idx_v, dst_v, b_v, g_v, acc, sem_ld, sem_g, sem_sc):
        cid = lax.axis_index("c")
        sid = lax.axis_index("s")
        wid = cid * NS + sid

        @pl.when(sid == 0)
        def _():
            pltpu.sync_copy(z_ref, acc)

        plsc.subcore_barrier()

        ebase = wid * EW

        @pl.loop(0, NCH)
        def _(i):
            base = ebase + i * CH
            cps = [pltpu.async_copy(gidx_ref.at[pl.ds(t * E + base, CH)],
                                    idx_v.at[t], sem_ld) for t in range(4)]
            cps.append(pltpu.async_copy(dst_ref.at[pl.ds(base, CH)],
                                        dst_v, sem_ld))
            cps.extend(pltpu.async_copy(b_ref.at[pl.ds(t * E + base, CH)],
                                        b_v.at[pl.ds(t * CH, CH)], sem_ld)
                       for t in range(4))
            for cp in cps:
                cp.wait()
            gcps = [pltpu.async_copy(t_ref.at[idx_v.at[t]], g_v.at[t], sem_g)
                    for t in range(4)]
            for cp in gcps:
                cp.wait()

            @pl.loop(0, CH)
            def _(e):
                eidx = jnp.full((L,), e, jnp.int32)
                bb = [plsc.load_gather(b_v, [eidx + (t * CH)])
                      for t in range(4)]
                for v in range(C // L):
                    a = g_v[0, e, pl.ds(v * L, L)] * bb[0]
                    a = a + g_v[1, e, pl.ds(v * L, L)] * bb[1]
                    a = a + g_v[2, e, pl.ds(v * L, L)] * bb[2]
                    a = a + g_v[3, e, pl.ds(v * L, L)] * bb[3]
                    g_v[0, e, pl.ds(v * L, L)] = a

            pltpu.async_copy(g_v.at[0], acc.at[dst_v], sem_sc,
                             add=True).wait()

        plsc.subcore_barrier()

        @pl.when(sid == 0)
        def _():
            pltpu.sync_copy(acc, out_ref.at[cid])

    return conv(t_flat, gidx, dst, b4, zeros)


def kernel(x, pos, edge_index, edge_attr, W1, Wr1, g1, b1,
           W2, Wr2, g2, b2, Wlin, g3, b3):
    xin = jnp.concatenate([x, pos[:, :2]], axis=1)
    rows = E // C
    src2d = edge_index[0].reshape(rows, C)
    ea0 = edge_attr[:, 0].reshape(rows, C)
    ea1 = edge_attr[:, 1].reshape(rows, C)
    gidx_r, b_r = _edge_prep(src2d, ea0, ea1)
    gidx = gidx_r.reshape(4 * E)
    b4 = b_r.reshape(4 * E)
    dst = edge_index[1]

    zeros = jnp.zeros((N, C), jnp.float32)
    cnt_r = _sc_cnt(dst)
    cnt = cnt_r.reshape(NC, NR * C)[:, :N, None]
    t1 = _t_build(xin, W1).reshape(K * N, C)
    acc1 = _sc_conv(t1, gidx, dst, b4, zeros)
    h = _post1(acc1, cnt, xin, Wr1, g1, b1)

    t2 = _t_build(h, W2).reshape(K * N, C)
    acc2 = _sc_conv(t2, gidx, dst, b4, zeros)
    return _post2(acc2, cnt, h, Wr2, xin, Wlin, g2, b2, g3, b3)


# speedup vs baseline: 3.2139x; 1.7121x over previous
import functools

import jax
import jax.numpy as jnp
from jax import lax
from jax.experimental import pallas as pl
from jax.experimental.pallas import tpu as pltpu
from jax.experimental.pallas import tpu_sc as plsc

N = 10000
E = 320000
FP = 130
C = 128
M = 5
K = M * M

NC, NS, L = 2, 16, 16
NW = NC * NS
EW = E // NW
CH = 40
NCH = EW // CH
RPT = N // NS
D1 = C + L


def _prep_body(src_ref, ea0_ref, ea1_ref, gidx_ref, b_ref):
    src = src_ref[...]
    v0 = ea0_ref[...] * float(M - 1)
    v1 = ea1_ref[...] * float(M - 1)
    i0 = jnp.clip(jnp.floor(v0), 0.0, float(M - 2))
    i1 = jnp.clip(jnp.floor(v1), 0.0, float(M - 2))
    f0 = v0 - i0
    f1 = v1 - i1
    base = (i0.astype(jnp.int32) * M + i1.astype(jnp.int32)) * N + src
    gidx_ref[0] = base
    gidx_ref[1] = base + N
    gidx_ref[2] = base + M * N
    gidx_ref[3] = base + (M + 1) * N
    b_ref[0] = (1.0 - f0) * (1.0 - f1)
    b_ref[1] = (1.0 - f0) * f1
    b_ref[2] = f0 * (1.0 - f1)
    b_ref[3] = f0 * f1


def _edge_prep(src2d, ea0, ea1):
    r, c = src2d.shape
    return pl.pallas_call(
        _prep_body,
        out_shape=[jax.ShapeDtypeStruct((4, r, c), jnp.int32),
                   jax.ShapeDtypeStruct((4, r, c), jnp.float32)],
    )(src2d, ea0, ea1)


def _tk_body(x_ref, w_ref, out_ref):
    out_ref[0] = jnp.dot(x_ref[...], w_ref[0],
                         preferred_element_type=jnp.float32)


def _t_build(x, W):
    k, f, c = W.shape
    n = x.shape[0]
    return pl.pallas_call(
        _tk_body,
        grid=(k,),
        in_specs=[pl.BlockSpec((n, f), lambda i: (0, 0)),
                  pl.BlockSpec((1, f, c), lambda i: (i, 0, 0))],
        out_specs=pl.BlockSpec((1, n, c), lambda i: (i, 0, 0)),
        out_shape=jax.ShapeDtypeStruct((k, n, c), jnp.float32),
    )(x, W)


def _bn(v, g, b):
    mu = jnp.mean(v, axis=0, keepdims=True)
    var = jnp.mean((v - mu) ** 2, axis=0, keepdims=True)
    return (v - mu) * lax.rsqrt(var + 1e-5) * g + b


def _post1_body(acc_ref, cnt_ref, xin_ref, wr_ref, g_ref, b_ref, h_ref):
    cnt = jnp.maximum(cnt_ref[0] + cnt_ref[1], 1.0)
    conv = (acc_ref[0] + acc_ref[1]) / cnt + jnp.dot(
        xin_ref[...], wr_ref[...], preferred_element_type=jnp.float32)
    h_ref[...] = jnp.maximum(_bn(conv, g_ref[...], b_ref[...]), 0.0)


def _post1(acc, cnt, xin, wr, g, b):
    return pl.pallas_call(
        _post1_body,
        out_shape=jax.ShapeDtypeStruct((N, C), jnp.float32),
    )(acc, cnt, xin, wr, g.reshape(1, C), b.reshape(1, C))


def _post2_body(acc_ref, cnt_ref, h_ref, wr_ref, xin_ref, wlin_ref,
                g2_ref, b2_ref, g3_ref, b3_ref, out_ref):
    cnt = jnp.maximum(cnt_ref[0] + cnt_ref[1], 1.0)
    conv = (acc_ref[0] + acc_ref[1]) / cnt + jnp.dot(
        h_ref[...], wr_ref[...], preferred_element_type=jnp.float32)
    y = _bn(conv, g2_ref[...], b2_ref[...])
    sk = _bn(jnp.dot(xin_ref[...], wlin_ref[...],
                     preferred_element_type=jnp.float32),
             g3_ref[...], b3_ref[...])
    out_ref[...] = jnp.maximum(y + sk, 0.0)


def _post2(acc2, cnt, h, wr2, xin, wlin, g2, b2, g3, b3):
    return pl.pallas_call(
        _post2_body,
        out_shape=jax.ShapeDtypeStruct((N, C), jnp.float32),
    )(acc2, cnt, h, wr2, xin, wlin,
      g2.reshape(1, C), b2.reshape(1, C), g3.reshape(1, C), b3.reshape(1, C))


NR = 80
CHD = 2000


def _sc_cnt(dst):
    mesh = plsc.VectorSubcoreMesh(core_axis_name="c", subcore_axis_name="s",
                                  num_cores=NC, num_subcores=NS)

    @functools.partial(
        pl.kernel,
        out_type=jax.ShapeDtypeStruct((NC, NR, C), jnp.float32),
        mesh=mesh,
        compiler_params=pltpu.CompilerParams(needs_layout_passes=False),
        scratch_types=[
            pltpu.VMEM((CHD,), jnp.int32),
            pltpu.VMEM((NR, C), jnp.float32),
            pltpu.VMEM((NR,), jnp.int32),
            pltpu.VMEM_SHARED((NR, C), jnp.float32),
            pltpu.SemaphoreType.DMA,
        ],
    )
    def cnt_k(dst_ref, cnt_out, dst_v, cnt_v, rid_v, cnt_sh, sem):
        cid = lax.axis_index("c")
        sid = lax.axis_index("s")
        wid = cid * NS + sid
        zv = jnp.zeros((L,), jnp.float32)
        iv = lax.iota(jnp.int32, L)
        ones = jnp.ones((L,), jnp.float32)

        @pl.loop(0, NR)
        def _(r):
            for v in range(C // L):
                cnt_v[r, pl.ds(v * L, L)] = zv

        @pl.loop(0, NR // L)
        def _(g):
            rid_v[pl.ds(g * L, L)] = iv + g * L

        @pl.when(sid == 0)
        def _():
            pltpu.sync_copy(cnt_v, cnt_sh)

        plsc.subcore_barrier()

        ebase = wid * EW

        @pl.loop(0, EW // CHD)
        def _(i):
            pltpu.async_copy(dst_ref.at[pl.ds(ebase + i * CHD, CHD)],
                             dst_v, sem).wait()

            @pl.loop(0, CHD // L)
            def _(g):
                dv = dst_v[pl.ds(g * L, L)]
                plsc.addupdate_scatter(
                    cnt_v, [lax.shift_right_logical(dv, 7),
                            jnp.bitwise_and(dv, 127)], ones)

        pltpu.async_copy(cnt_v, cnt_sh.at[rid_v], sem, add=True).wait()
        plsc.subcore_barrier()

        @pl.when(sid == 0)
        def _():
            pltpu.sync_copy(cnt_sh, cnt_out.at[cid])

    return cnt_k(dst)


def _sc_conv(t_flat, gidx, dst, b4, zeros):
    mesh = plsc.VectorSubcoreMesh(core_axis_name="c", subcore_axis_name="s",
                                  num_cores=NC, num_subcores=NS)

    @functools.partial(
        pl.kernel,
        out_type=jax.ShapeDtypeStruct((NC, N, C), jnp.float32),
        mesh=mesh,
        compiler_params=pltpu.CompilerParams(needs_layout_passes=False),
        scratch_types=[
            pltpu.VMEM((4, CH), jnp.int32),
            pltpu.VMEM((4, CH), jnp.int32),
            pltpu.VMEM((CH,), jnp.int32),
            pltpu.VMEM((CH,), jnp.int32),
            pltpu.VMEM((CH,), jnp.int32),
            pltpu.VMEM((CH,), jnp.int32),
            pltpu.VMEM((4 * CH,), jnp.float32),
            pltpu.VMEM((4 * CH,), jnp.float32),
            pltpu.VMEM((4, CH, C), jnp.float32),
            pltpu.VMEM((4, CH, C), jnp.float32),
            pltpu.VMEM_SHARED((N, C), jnp.float32),
            pltpu.SemaphoreType.DMA,
            pltpu.SemaphoreType.DMA,
            pltpu.SemaphoreType.DMA,
        ],
    )
    def conv(t_ref, gidx_ref, dst_ref, b_ref, z_ref, out_ref,
             idx_v0, idx_v1, dst_v0, dst_v1, dsc_v0, dsc_v1, b_v0, b_v1,
             g_v0, g_v1, acc, sem_ld, sem_g, sem_sc):
        idx_v = (idx_v0, idx_v1)
        dst_v = (dst_v0, dst_v1)
        dsc_v = (dsc_v0, dsc_v1)
        b_v = (b_v0, b_v1)
        g_v = (g_v0, g_v1)
        cid = lax.axis_index("c")
        sid = lax.axis_index("s")
        wid = cid * NS + sid
        ebase = wid * EW

        def fire_smalls(i, s):
            base = ebase + i * CH
            for t in range(4):
                pltpu.async_copy(gidx_ref.at[pl.ds(t * E + base, CH)],
                                 idx_v[s].at[t], sem_ld)
            pltpu.async_copy(dst_ref.at[pl.ds(base, CH)],
                             dst_v[s], sem_ld)
            for t in range(4):
                pltpu.async_copy(b_ref.at[pl.ds(t * E + base, CH)],
                                 b_v[s].at[pl.ds(t * CH, CH)], sem_ld)

        def wait_smalls(s):
            for t in range(4):
                pltpu.make_async_copy(gidx_ref.at[pl.ds(0, CH)],
                                      idx_v[s].at[t], sem_ld).wait()
            pltpu.make_async_copy(dst_ref.at[pl.ds(0, CH)],
                                  dst_v[s], sem_ld).wait()
            for t in range(4):
                pltpu.make_async_copy(b_ref.at[pl.ds(0, CH)],
                                      b_v[s].at[pl.ds(t * CH, CH)],
                                      sem_ld).wait()

        def fire_gathers(s):
            for t in range(4):
                pltpu.async_copy(t_ref.at[idx_v[s].at[t]], g_v[s].at[t],
                                 sem_g)

        def wait_gathers(s):
            for t in range(4):
                pltpu.make_async_copy(t_ref.at[idx_v[s].at[t]],
                                      g_v[s].at[t], sem_g).wait()

        def fire_scatter(s):
            pltpu.async_copy(g_v[s].at[0], acc.at[dsc_v[s]], sem_sc,
                             add=True)

        def wait_scatter(s):
            pltpu.make_async_copy(g_v[s].at[0], acc.at[dsc_v[s]],
                                  sem_sc).wait()

        def compute(s):
            gv = g_v[s]
            bv = b_v[s]

            @pl.loop(0, CH)
            def _(e):
                eidx = jnp.full((L,), e, jnp.int32)
                bb = [plsc.load_gather(bv, [eidx + (t * CH)])
                      for t in range(4)]
                for v in range(C // L):
                    a = gv[0, e, pl.ds(v * L, L)] * bb[0]
                    a = a + gv[1, e, pl.ds(v * L, L)] * bb[1]
                    a = a + gv[2, e, pl.ds(v * L, L)] * bb[2]
                    a = a + gv[3, e, pl.ds(v * L, L)] * bb[3]
                    gv[0, e, pl.ds(v * L, L)] = a

            for off in (0, 16, 24):
                dsc_v[s][pl.ds(off, L)] = dst_v[s][pl.ds(off, L)]

        @pl.when(sid == 0)
        def _():
            pltpu.sync_copy(z_ref, acc)

        plsc.subcore_barrier()

        fire_smalls(0, 0)
        wait_smalls(0)
        fire_gathers(0)
        fire_smalls(1, 1)

        @pl.loop(0, NCH // 2)
        def _(j):
            for ph in range(2):
                i = 2 * j + ph
                s, o = ph, 1 - ph
                wait_gathers(s)

                @pl.when(i > 0)
                def _():
                    wait_scatter(o)

                @pl.when(i < NCH - 1)
                def _():
                    wait_smalls(o)
                    fire_gathers(o)

                compute(s)
                fire_scatter(s)

                @pl.when(i < NCH - 2)
                def _():
                    fire_smalls(i + 2, s)

        wait_scatter((NCH - 1) % 2)
        plsc.subcore_barrier()

        @pl.when(sid == 0)
        def _():
            pltpu.sync_copy(acc, out_ref.at[cid])

    return conv(t_flat, gidx, dst, b4, zeros)


def kernel(x, pos, edge_index, edge_attr, W1, Wr1, g1, b1,
           W2, Wr2, g2, b2, Wlin, g3, b3):
    xin = jnp.concatenate([x, pos[:, :2]], axis=1)
    rows = E // C
    src2d = edge_index[0].reshape(rows, C)
    ea0 = edge_attr[:, 0].reshape(rows, C)
    ea1 = edge_attr[:, 1].reshape(rows, C)
    gidx_r, b_r = _edge_prep(src2d, ea0, ea1)
    gidx = gidx_r.reshape(4 * E)
    b4 = b_r.reshape(4 * E)
    dst = edge_index[1]

    zeros = jnp.zeros((N, C), jnp.float32)
    cnt_r = _sc_cnt(dst)
    cnt = cnt_r.reshape(NC, NR * C)[:, :N, None]
    t1 = _t_build(xin, W1).reshape(K * N, C)
    acc1 = _sc_conv(t1, gidx, dst, b4, zeros)
    h = _post1(acc1, cnt, xin, Wr1, g1, b1)

    t2 = _t_build(h, W2).reshape(K * N, C)
    acc2 = _sc_conv(t2, gidx, dst, b4, zeros)
    return _post2(acc2, cnt, h, Wr2, xin, Wlin, g2, b2, g3, b3)
